# Initial kernel scaffold; baseline (speedup 1.0000x reference)
import jax
import jax.numpy as jnp
from jax.experimental import pallas as pl


def kernel(noised_data, t, edge_index, W_in, b_in, W0, b0, W1, b1, W2, b2, p_w, p_b, gamma, beta, W_fc, b_fc):
    raise NotImplementedError("write your pallas kernel here")


# speedup vs baseline: 3.6719x; 3.6719x over previous
import functools
import math

import jax
import jax.numpy as jnp
from jax import lax
from jax.experimental import pallas as pl
from jax.experimental.pallas import tpu as pltpu
from jax.experimental.pallas import tpu_sc as plsc

F32 = jnp.float32
I32 = jnp.int32
NW = 32
NTILE = 16
C = 128


def _round_up(x, m):
    return (x + m - 1) // m * m


CO = 64


@functools.cache
def _edge_pass(n_out_pad, e_pad, d):
    w_per = e_pad // (NW * C)
    mesh = plsc.VectorSubcoreMesh(core_axis_name="c", subcore_axis_name="s")
    rpt = n_out_pad // NTILE
    assert rpt % CO == 0

    @functools.partial(
        pl.kernel,
        out_type=jax.ShapeDtypeStruct((2, n_out_pad, d), F32),
        mesh=mesh,
        scratch_types=[
            pltpu.VMEM((C,), I32),
            pltpu.VMEM((C,), I32),
            pltpu.VMEM((C, d), F32),
            pltpu.VMEM((CO, d), F32),
            pltpu.VMEM_SHARED((n_out_pad, d), F32),
            pltpu.SemaphoreType.DMA,
        ],
    )
    def k(hs, srcp, dstp, out, src_v, dst_v, rows_v, stg_v, acc, sem):
        c = lax.axis_index("c")
        s = lax.axis_index("s")
        wid = c * NTILE + s

        def zfill(i, carry):
            stg_v[i // jnp.int32(d // 16),
                  pl.ds((i % jnp.int32(d // 16)) * 16, 16)] = (
                      jnp.zeros((16,), F32))
            return carry
        lax.fori_loop(jnp.int32(0), jnp.int32(CO * d // 16), zfill,
                      jnp.int32(0))

        def zcp(i, carry):
            pltpu.sync_copy(stg_v, acc.at[pl.ds(s * rpt + i * jnp.int32(CO),
                                                CO)])
            return carry
        lax.fori_loop(jnp.int32(0), jnp.int32(rpt // CO), zcp, jnp.int32(0))
        plsc.subcore_barrier()

        def body(w, carry):
            base = (wid * jnp.int32(w_per) + w) * jnp.int32(C)
            pltpu.sync_copy(srcp.at[pl.ds(base, C)], src_v)
            pltpu.sync_copy(dstp.at[pl.ds(base, C)], dst_v)
            pltpu.async_copy(hs.at[src_v], rows_v, sem).wait()
            pltpu.sync_copy(rows_v, acc.at[dst_v], add=True)
            return carry

        lax.fori_loop(jnp.int32(0), jnp.int32(w_per), body, jnp.int32(0))
        plsc.subcore_barrier()

        def ocp(i, carry):
            off = s * rpt + i * jnp.int32(CO)
            pltpu.sync_copy(acc.at[pl.ds(off, CO)], stg_v)
            pltpu.sync_copy(stg_v, out.at[c, pl.ds(off, CO)])
            return carry
        lax.fori_loop(jnp.int32(0), jnp.int32(rpt // CO), ocp, jnp.int32(0))

    return k


@functools.cache
def _deg_pass(n_out_pad, e_pad):
    w_per = e_pad // (NW * C)
    mesh = plsc.VectorSubcoreMesh(core_axis_name="c", subcore_axis_name="s")
    rpt = n_out_pad // NTILE

    @functools.partial(
        pl.kernel,
        out_type=jax.ShapeDtypeStruct((2 * n_out_pad,), F32),
        mesh=mesh,
        scratch_types=[
            pltpu.VMEM((C,), I32),
            pltpu.VMEM((C,), F32),
            pltpu.VMEM((rpt,), F32),
            pltpu.VMEM_SHARED((n_out_pad,), F32),
        ],
    )
    def k(dstp, out, dst_v, ones_v, stg_v, acc):
        c = lax.axis_index("c")
        s = lax.axis_index("s")
        wid = c * NTILE + s
        for i in range(C // 16):
            ones_v[pl.ds(i * 16, 16)] = jnp.full((16,), 1.0, F32)

        def zfill(i, carry):
            stg_v[pl.ds(i * 16, 16)] = jnp.zeros((16,), F32)
            return carry
        lax.fori_loop(jnp.int32(0), jnp.int32(rpt // 16), zfill, jnp.int32(0))
        pltpu.sync_copy(stg_v, acc.at[pl.ds(s * rpt, rpt)])
        plsc.subcore_barrier()

        def body(w, carry):
            base = (wid * jnp.int32(w_per) + w) * jnp.int32(C)
            pltpu.sync_copy(dstp.at[pl.ds(base, C)], dst_v)
            pltpu.sync_copy(ones_v, acc.at[dst_v], add=True)
            return carry

        lax.fori_loop(jnp.int32(0), jnp.int32(w_per), body, jnp.int32(0))
        plsc.subcore_barrier()
        pltpu.sync_copy(acc.at[pl.ds(s * rpt, rpt)], stg_v)
        pltpu.sync_copy(
            stg_v, out.at[pl.ds(c * jnp.int32(n_out_pad) + s * rpt, rpt)])

    return k


def _pad_edges(src, dst, n_in, n_out):
    e = src.shape[0]
    e_pad = _round_up(e, NW * C)
    pad = e_pad - e
    i = jnp.arange(pad, dtype=I32)
    src_p = jnp.concatenate([src, i % jnp.int32(n_in)])
    dst_p = jnp.concatenate([dst, jnp.int32(n_out) + (i % 64)])
    return src_p, dst_p, e_pad


def _sinus_row(t, dim):
    half = dim // 2
    cst = math.log(10000.0) / (half - 1)
    freqs = jnp.exp(jnp.arange(half, dtype=F32) * (-cst))
    e = t[0].astype(F32) * freqs
    return jnp.concatenate([jnp.sin(e), jnp.cos(e)])


def kernel(noised_data, t, edge_index, W_in, b_in, W0, b0, W1, b1, W2, b2,
           p_w, p_b, gamma, beta, W_fc, b_fc):
    n = noised_data.shape[1]
    kk = n // 2
    d = W0.shape[1]
    ei = edge_index.astype(I32)
    src0, dst0 = ei[0], ei[1]
    e = src0.shape[0]

    pad_n = _round_up(n + 64, 1024)
    pad_k = _round_up(kk + 64, 1024)

    src0p, dst0p, e_pad = _pad_edges(src0, dst0, n, n)

    dp = _deg_pass(pad_n, e_pad)(dst0p)
    deg0 = dp[:n] + dp[pad_n:pad_n + n] + 1.0
    r0 = lax.rsqrt(deg0)

    x = noised_data[0] @ W_in + b_in
    temb = _sinus_row(t, d)
    const0 = temb @ W0[W_in.shape[1]:]
    H0 = x @ W0[: W_in.shape[1]] + const0
    Hs0 = H0 * r0[:, None]

    S = _edge_pass(pad_n, e_pad, d)(Hs0, src0p, dst0p)
    h0 = jax.nn.relu(r0[:, None] * (S[0, :n] + S[1, :n] + Hs0) + b0)

    y = (h0 @ p_w)[:, 0] + p_b[0]
    topv, idx = lax.top_k(y, kk)
    idx32 = idx.astype(I32)
    yh = jax.nn.sigmoid(topv)
    x1 = h0[idx32] * yh[:, None]

    newid = jnp.full((n,), -1, I32).at[idx32].set(
        jnp.arange(kk, dtype=I32))
    s1 = newid[src0]
    d1 = newid[dst0]
    m = (s1 >= 0) & (d1 >= 0)
    key = jnp.where(m, s1 * kk + d1, 0)
    eid = jnp.arange(e, dtype=I32)
    table = jnp.zeros((kk * kk,), I32).at[key].set(eid, mode="drop")
    valid = m & (table[key] == eid)

    iv = jnp.arange(e, dtype=I32)
    s1s = jnp.where(valid, s1, iv % jnp.int32(kk))
    d1s = jnp.where(valid, d1, jnp.int32(kk) + (iv % 64))
    s1p, d1p, _ = _pad_edges(s1s, d1s, kk, kk)

    dp1 = _deg_pass(pad_k, e_pad)(d1p)
    deg1 = dp1[:kk] + dp1[pad_k:pad_k + kk] + 1.0
    r1 = lax.rsqrt(deg1)
    H1 = x1 @ W1
    Hs1 = H1 * r1[:, None]
    S1 = _edge_pass(pad_k, e_pad, d)(Hs1, s1p, d1p)
    h1 = jax.nn.relu(r1[:, None] * (S1[0, :kk] + S1[1, :kk] + Hs1) + b1)

    A = h0 @ W2
    B = h1 @ W2
    usrc, udst, ue_pad = _pad_edges(jnp.arange(kk, dtype=I32), idx32, kk, n)
    SU = _edge_pass(pad_n, ue_pad, d)(B, usrc, udst)
    U2 = SU[0, :n] + SU[1, :n] + A
    Hs2 = U2 * r0[:, None]

    S2 = _edge_pass(pad_n, e_pad, d)(Hs2, src0p, dst0p)
    h2 = jax.nn.relu(r0[:, None] * (S2[0, :n] + S2[1, :n] + Hs2) + b2)

    mu = jnp.mean(h2, axis=-1, keepdims=True)
    var = jnp.mean((h2 - mu) ** 2, axis=-1, keepdims=True)
    h2 = (h2 - mu) / jnp.sqrt(var + 1e-5) * gamma + beta
    return (h2 @ W_fc + b_fc)[None, ...]


# speedup vs baseline: 5.3947x; 1.4692x over previous
import functools
import math

import jax
import jax.numpy as jnp
from jax import lax
from jax.experimental import pallas as pl
from jax.experimental.pallas import tpu as pltpu
from jax.experimental.pallas import tpu_sc as plsc

F32 = jnp.float32
I32 = jnp.int32
NW = 32
NTILE = 16
C = 128


def _round_up(x, m):
    return (x + m - 1) // m * m


CO = 64


@functools.cache
def _edge_pass(n_out_pad, e_pad, d):
    w_per = e_pad // (NW * C)
    mesh = plsc.VectorSubcoreMesh(core_axis_name="c", subcore_axis_name="s")
    rpt = n_out_pad // NTILE
    assert rpt % CO == 0

    @functools.partial(
        pl.kernel,
        out_type=jax.ShapeDtypeStruct((2, n_out_pad, d), F32),
        mesh=mesh,
        scratch_types=[
            pltpu.VMEM((C,), I32),
            pltpu.VMEM((C,), I32),
            pltpu.VMEM((C, d), F32),
            pltpu.VMEM((CO, d), F32),
            pltpu.VMEM_SHARED((n_out_pad, d), F32),
            pltpu.SemaphoreType.DMA,
        ],
    )
    def k(hs, srcp, dstp, out, src_v, dst_v, rows_v, stg_v, acc, sem):
        c = lax.axis_index("c")
        s = lax.axis_index("s")
        wid = c * NTILE + s

        def zfill(i, carry):
            stg_v[i // jnp.int32(d // 16),
                  pl.ds((i % jnp.int32(d // 16)) * 16, 16)] = (
                      jnp.zeros((16,), F32))
            return carry
        lax.fori_loop(jnp.int32(0), jnp.int32(CO * d // 16), zfill,
                      jnp.int32(0))

        def zcp(i, carry):
            pltpu.sync_copy(stg_v, acc.at[pl.ds(s * rpt + i * jnp.int32(CO),
                                                CO)])
            return carry
        lax.fori_loop(jnp.int32(0), jnp.int32(rpt // CO), zcp, jnp.int32(0))
        plsc.subcore_barrier()

        def body(w, carry):
            base = (wid * jnp.int32(w_per) + w) * jnp.int32(C)
            pltpu.sync_copy(srcp.at[pl.ds(base, C)], src_v)
            pltpu.sync_copy(dstp.at[pl.ds(base, C)], dst_v)
            pltpu.async_copy(hs.at[src_v], rows_v, sem).wait()
            pltpu.sync_copy(rows_v, acc.at[dst_v], add=True)
            return carry

        lax.fori_loop(jnp.int32(0), jnp.int32(w_per), body, jnp.int32(0))
        plsc.subcore_barrier()

        def ocp(i, carry):
            off = s * rpt + i * jnp.int32(CO)
            pltpu.sync_copy(acc.at[pl.ds(off, CO)], stg_v)
            pltpu.sync_copy(stg_v, out.at[c, pl.ds(off, CO)])
            return carry
        lax.fori_loop(jnp.int32(0), jnp.int32(rpt // CO), ocp, jnp.int32(0))

    return k


@functools.cache
def _deg_pass(n_out_pad, e_pad):
    w_per = e_pad // (NW * C)
    mesh = plsc.VectorSubcoreMesh(core_axis_name="c", subcore_axis_name="s")
    rpt = n_out_pad // NTILE

    @functools.partial(
        pl.kernel,
        out_type=jax.ShapeDtypeStruct((2 * n_out_pad,), F32),
        mesh=mesh,
        scratch_types=[
            pltpu.VMEM((C,), I32),
            pltpu.VMEM((C,), F32),
            pltpu.VMEM((rpt,), F32),
            pltpu.VMEM_SHARED((n_out_pad,), F32),
        ],
    )
    def k(dstp, out, dst_v, ones_v, stg_v, acc):
        c = lax.axis_index("c")
        s = lax.axis_index("s")
        wid = c * NTILE + s
        for i in range(C // 16):
            ones_v[pl.ds(i * 16, 16)] = jnp.full((16,), 1.0, F32)

        def zfill(i, carry):
            stg_v[pl.ds(i * 16, 16)] = jnp.zeros((16,), F32)
            return carry
        lax.fori_loop(jnp.int32(0), jnp.int32(rpt // 16), zfill, jnp.int32(0))
        pltpu.sync_copy(stg_v, acc.at[pl.ds(s * rpt, rpt)])
        plsc.subcore_barrier()

        def body(w, carry):
            base = (wid * jnp.int32(w_per) + w) * jnp.int32(C)
            pltpu.sync_copy(dstp.at[pl.ds(base, C)], dst_v)
            pltpu.sync_copy(ones_v, acc.at[dst_v], add=True)
            return carry

        lax.fori_loop(jnp.int32(0), jnp.int32(w_per), body, jnp.int32(0))
        plsc.subcore_barrier()
        pltpu.sync_copy(acc.at[pl.ds(s * rpt, rpt)], stg_v)
        pltpu.sync_copy(
            stg_v, out.at[pl.ds(c * jnp.int32(n_out_pad) + s * rpt, rpt)])

    return k


def _iota16():
    return lax.iota(I32, 16)


@functools.cache
def _map_edges(n_pad, k_pad, e_pad, kk, tbl):
    w_map = k_pad // (NTILE * C)
    w_per = e_pad // (NW * C)
    mesh = plsc.VectorSubcoreMesh(core_axis_name="c", subcore_axis_name="s")
    rpt = n_pad // NTILE
    ktrash = kk * kk

    @functools.partial(
        pl.kernel,
        out_type=(jax.ShapeDtypeStruct((e_pad,), I32),
                  jax.ShapeDtypeStruct((e_pad,), I32),
                  jax.ShapeDtypeStruct((e_pad,), I32),
                  jax.ShapeDtypeStruct((tbl,), I32)),
        mesh=mesh,
        scratch_types=[
            pltpu.VMEM((C,), I32),
            pltpu.VMEM((C,), I32),
            pltpu.VMEM((C,), I32),
            pltpu.VMEM((C,), I32),
            pltpu.VMEM((C,), I32),
            pltpu.VMEM((C,), I32),
            pltpu.VMEM((rpt,), I32),
            pltpu.VMEM_SHARED((n_pad,), I32),
        ],
    )
    def k(idxp, srcp, dstp, s1o, d1o, keyo, tblo,
          a_v, b_v, s_v, d_v, key_v, eid_v, stg_v, newid):
        c = lax.axis_index("c")
        s = lax.axis_index("s")
        wid = c * NTILE + s

        def ifill(i, carry):
            stg_v[pl.ds(i * 16, 16)] = jnp.full((16,), -1, I32)
            return carry
        lax.fori_loop(jnp.int32(0), jnp.int32(rpt // 16), ifill, jnp.int32(0))
        pltpu.sync_copy(stg_v, newid.at[pl.ds(s * rpt, rpt)])
        plsc.subcore_barrier()

        def mbody(w, carry):
            base = (s * jnp.int32(w_map) + w) * jnp.int32(C)
            pltpu.sync_copy(idxp.at[pl.ds(base, C)], a_v)
            for j in range(C // 16):
                eid_v[pl.ds(j * 16, 16)] = base + jnp.int32(j * 16) + _iota16()
            pltpu.sync_copy(eid_v, newid.at[a_v])
            return carry
        lax.fori_loop(jnp.int32(0), jnp.int32(w_map), mbody, jnp.int32(0))
        plsc.subcore_barrier()

        def body(w, carry):
            base = (wid * jnp.int32(w_per) + w) * jnp.int32(C)
            pltpu.sync_copy(srcp.at[pl.ds(base, C)], a_v)
            pltpu.sync_copy(dstp.at[pl.ds(base, C)], b_v)
            pltpu.sync_copy(newid.at[a_v], s_v)
            pltpu.sync_copy(newid.at[b_v], d_v)
            for j in range(C // 16):
                sl = pl.ds(j * 16, 16)
                s16 = s_v[sl]
                d16 = d_v[sl]
                eid16 = base + jnp.int32(j * 16) + _iota16()
                m16 = (s16 >= 0) & (d16 >= 0)
                key16 = jnp.where(m16, s16 * jnp.int32(kk) + d16,
                                  jnp.int32(ktrash) + (eid16 & 2047))
                key_v[sl] = key16
                eid_v[sl] = eid16
            pltpu.sync_copy(s_v, s1o.at[pl.ds(base, C)])
            pltpu.sync_copy(d_v, d1o.at[pl.ds(base, C)])
            pltpu.sync_copy(key_v, keyo.at[pl.ds(base, C)])
            pltpu.sync_copy(eid_v, tblo.at[key_v])
            return carry
        lax.fori_loop(jnp.int32(0), jnp.int32(w_per), body, jnp.int32(0))

    return k


@functools.cache
def _finish_edges(pad_k, e_pad, kk):
    w_per = e_pad // (NW * C)
    mesh = plsc.VectorSubcoreMesh(core_axis_name="c", subcore_axis_name="s")
    rpt = pad_k // NTILE

    @functools.partial(
        pl.kernel,
        out_type=(jax.ShapeDtypeStruct((e_pad,), I32),
                  jax.ShapeDtypeStruct((e_pad,), I32),
                  jax.ShapeDtypeStruct((2 * pad_k,), F32)),
        mesh=mesh,
        scratch_types=[
            pltpu.VMEM((C,), I32),
            pltpu.VMEM((C,), I32),
            pltpu.VMEM((C,), I32),
            pltpu.VMEM((C,), I32),
            pltpu.VMEM((C,), I32),
            pltpu.VMEM((C,), I32),
            pltpu.VMEM((C,), F32),
            pltpu.VMEM((rpt,), F32),
            pltpu.VMEM_SHARED((pad_k,), F32),
            pltpu.SemaphoreType.DMA,
        ],
    )
    def k(s1, d1, key, tbl, srcfo, dstfo, dego,
          s_v, d_v, key_v, t_v, sf_v, df_v, ones_v, stg_v, acc, sem):
        c = lax.axis_index("c")
        s = lax.axis_index("s")
        wid = c * NTILE + s
        for i in range(C // 16):
            ones_v[pl.ds(i * 16, 16)] = jnp.full((16,), 1.0, F32)

        def zfill(i, carry):
            stg_v[pl.ds(i * 16, 16)] = jnp.zeros((16,), F32)
            return carry
        lax.fori_loop(jnp.int32(0), jnp.int32(rpt // 16), zfill, jnp.int32(0))
        pltpu.sync_copy(stg_v, acc.at[pl.ds(s * rpt, rpt)])
        plsc.subcore_barrier()

        def body(w, carry):
            base = (wid * jnp.int32(w_per) + w) * jnp.int32(C)
            pltpu.sync_copy(s1.at[pl.ds(base, C)], s_v)
            pltpu.sync_copy(d1.at[pl.ds(base, C)], d_v)
            pltpu.sync_copy(key.at[pl.ds(base, C)], key_v)
            pltpu.async_copy(tbl.at[key_v], t_v, sem).wait()
            for j in range(C // 16):
                sl = pl.ds(j * 16, 16)
                s16 = s_v[sl]
                d16 = d_v[sl]
                eid16 = base + jnp.int32(j * 16) + _iota16()
                ok = (s16 >= 0) & (d16 >= 0) & (t_v[sl] == eid16)
                sf_v[sl] = jnp.where(ok, s16, eid16 & 4095)
                df_v[sl] = jnp.where(ok, d16,
                                     jnp.int32(kk) + (eid16 & 63))
            pltpu.sync_copy(sf_v, srcfo.at[pl.ds(base, C)])
            pltpu.sync_copy(df_v, dstfo.at[pl.ds(base, C)])
            pltpu.sync_copy(ones_v, acc.at[df_v], add=True)
            return carry
        lax.fori_loop(jnp.int32(0), jnp.int32(w_per), body, jnp.int32(0))
        plsc.subcore_barrier()
        pltpu.sync_copy(acc.at[pl.ds(s * rpt, rpt)], stg_v)
        pltpu.sync_copy(
            stg_v, dego.at[pl.ds(c * jnp.int32(pad_k) + s * rpt, rpt)])

    return k


def _pad_edges(src, dst, n_in, n_out):
    e = src.shape[0]
    e_pad = _round_up(e, NW * C)
    pad = e_pad - e
    i = jnp.arange(pad, dtype=I32)
    src_p = jnp.concatenate([src, i % jnp.int32(n_in)])
    dst_p = jnp.concatenate([dst, jnp.int32(n_out) + (i % 64)])
    return src_p, dst_p, e_pad


def _sinus_row(t, dim):
    half = dim // 2
    cst = math.log(10000.0) / (half - 1)
    freqs = jnp.exp(jnp.arange(half, dtype=F32) * (-cst))
    e = t[0].astype(F32) * freqs
    return jnp.concatenate([jnp.sin(e), jnp.cos(e)])


def kernel(noised_data, t, edge_index, W_in, b_in, W0, b0, W1, b1, W2, b2,
           p_w, p_b, gamma, beta, W_fc, b_fc):
    n = noised_data.shape[1]
    kk = n // 2
    d = W0.shape[1]
    ei = edge_index.astype(I32)
    src0, dst0 = ei[0], ei[1]
    e = src0.shape[0]

    pad_n = _round_up(n + 64, 1024)
    pad_k = _round_up(kk + 64, 1024)

    src0p, dst0p, e_pad = _pad_edges(src0, dst0, n, n)

    dp = _deg_pass(pad_n, e_pad)(dst0p)
    deg0 = dp[:n] + dp[pad_n:pad_n + n] + 1.0
    r0 = lax.rsqrt(deg0)

    x = noised_data[0] @ W_in + b_in
    temb = _sinus_row(t, d)
    const0 = temb @ W0[W_in.shape[1]:]
    H0 = x @ W0[: W_in.shape[1]] + const0
    Hs0 = H0 * r0[:, None]

    S = _edge_pass(pad_n, e_pad, d)(Hs0, src0p, dst0p)
    h0 = jax.nn.relu(r0[:, None] * (S[0, :n] + S[1, :n] + Hs0) + b0)

    y = (h0 @ p_w)[:, 0] + p_b[0]
    topv, idx = lax.top_k(y, kk)
    idx32 = idx.astype(I32)
    yh = jax.nn.sigmoid(topv)
    x1 = h0[idx32] * yh[:, None]

    k_pad = _round_up(kk, NTILE * C)
    idxp = jnp.concatenate([
        idx32,
        jnp.int32(n + 64) + (jnp.arange(k_pad - kk, dtype=I32) % 64)])
    tbl = kk * kk + 2048
    s1a, d1a, keya, tbla = _map_edges(pad_n, k_pad, e_pad, kk, tbl)(
        idxp, src0p, dst0p)
    s1p, d1p, dego1 = _finish_edges(pad_k, e_pad, kk)(s1a, d1a, keya, tbla)
    deg1 = dego1[:kk] + dego1[pad_k:pad_k + kk] + 1.0
    r1 = lax.rsqrt(deg1)
    H1 = x1 @ W1
    Hs1 = H1 * r1[:, None]
    S1 = _edge_pass(pad_k, e_pad, d)(Hs1, s1p, d1p)
    h1 = jax.nn.relu(r1[:, None] * (S1[0, :kk] + S1[1, :kk] + Hs1) + b1)

    A = h0 @ W2
    B = h1 @ W2
    usrc, udst, ue_pad = _pad_edges(jnp.arange(kk, dtype=I32), idx32, kk, n)
    SU = _edge_pass(pad_n, ue_pad, d)(B, usrc, udst)
    U2 = SU[0, :n] + SU[1, :n] + A
    Hs2 = U2 * r0[:, None]

    S2 = _edge_pass(pad_n, e_pad, d)(Hs2, src0p, dst0p)
    h2 = jax.nn.relu(r0[:, None] * (S2[0, :n] + S2[1, :n] + Hs2) + b2)

    mu = jnp.mean(h2, axis=-1, keepdims=True)
    var = jnp.mean((h2 - mu) ** 2, axis=-1, keepdims=True)
    h2 = (h2 - mu) / jnp.sqrt(var + 1e-5) * gamma + beta
    return (h2 @ W_fc + b_fc)[None, ...]


# speedup vs baseline: 5.4033x; 1.0016x over previous
import functools
import math

import jax
import jax.numpy as jnp
from jax import lax
from jax.experimental import pallas as pl
from jax.experimental.pallas import tpu as pltpu
from jax.experimental.pallas import tpu_sc as plsc

F32 = jnp.float32
I32 = jnp.int32
NW = 32
NTILE = 16
C = 128


def _round_up(x, m):
    return (x + m - 1) // m * m


CO = 64


@functools.cache
def _edge_pass(n_out_pad, e_pad, d):
    w_per = e_pad // (NW * C)
    mesh = plsc.VectorSubcoreMesh(core_axis_name="c", subcore_axis_name="s")
    rpt = n_out_pad // NTILE
    assert rpt % CO == 0

    @functools.partial(
        pl.kernel,
        name="edge_pass",
        out_type=jax.ShapeDtypeStruct((2, n_out_pad, d), F32),
        mesh=mesh,
        scratch_types=[
            pltpu.VMEM((C,), I32),
            pltpu.VMEM((C,), I32),
            pltpu.VMEM((C, d), F32),
            pltpu.VMEM((CO, d), F32),
            pltpu.VMEM_SHARED((n_out_pad, d), F32),
            pltpu.SemaphoreType.DMA,
        ],
    )
    def k(hs, srcp, dstp, out, src_v, dst_v, rows_v, stg_v, acc, sem):
        c = lax.axis_index("c")
        s = lax.axis_index("s")
        wid = c * NTILE + s

        def zfill(i, carry):
            stg_v[i // jnp.int32(d // 16),
                  pl.ds((i % jnp.int32(d // 16)) * 16, 16)] = (
                      jnp.zeros((16,), F32))
            return carry
        lax.fori_loop(jnp.int32(0), jnp.int32(CO * d // 16), zfill,
                      jnp.int32(0))

        def zcp(i, carry):
            pltpu.sync_copy(stg_v, acc.at[pl.ds(s * rpt + i * jnp.int32(CO),
                                                CO)])
            return carry
        lax.fori_loop(jnp.int32(0), jnp.int32(rpt // CO), zcp, jnp.int32(0))
        plsc.subcore_barrier()

        def body(w, carry):
            base = (wid * jnp.int32(w_per) + w) * jnp.int32(C)
            pltpu.sync_copy(srcp.at[pl.ds(base, C)], src_v)
            pltpu.sync_copy(dstp.at[pl.ds(base, C)], dst_v)
            pltpu.async_copy(hs.at[src_v], rows_v, sem).wait()
            pltpu.sync_copy(rows_v, acc.at[dst_v], add=True)
            return carry

        lax.fori_loop(jnp.int32(0), jnp.int32(w_per), body, jnp.int32(0))
        plsc.subcore_barrier()

        def ocp(i, carry):
            off = s * rpt + i * jnp.int32(CO)
            pltpu.sync_copy(acc.at[pl.ds(off, CO)], stg_v)
            pltpu.sync_copy(stg_v, out.at[c, pl.ds(off, CO)])
            return carry
        lax.fori_loop(jnp.int32(0), jnp.int32(rpt // CO), ocp, jnp.int32(0))

    return k


@functools.cache
def _deg_pass(n_out_pad, e_pad):
    w_per = e_pad // (NW * C)
    mesh = plsc.VectorSubcoreMesh(core_axis_name="c", subcore_axis_name="s")
    rpt = n_out_pad // NTILE

    @functools.partial(
        pl.kernel,
        name="deg_pass",
        out_type=jax.ShapeDtypeStruct((2 * n_out_pad,), F32),
        mesh=mesh,
        scratch_types=[
            pltpu.VMEM((C,), I32),
            pltpu.VMEM((C,), F32),
            pltpu.VMEM((rpt,), F32),
            pltpu.VMEM_SHARED((n_out_pad,), F32),
        ],
    )
    def k(dstp, out, dst_v, ones_v, stg_v, acc):
        c = lax.axis_index("c")
        s = lax.axis_index("s")
        wid = c * NTILE + s
        for i in range(C // 16):
            ones_v[pl.ds(i * 16, 16)] = jnp.full((16,), 1.0, F32)

        def zfill(i, carry):
            stg_v[pl.ds(i * 16, 16)] = jnp.zeros((16,), F32)
            return carry
        lax.fori_loop(jnp.int32(0), jnp.int32(rpt // 16), zfill, jnp.int32(0))
        pltpu.sync_copy(stg_v, acc.at[pl.ds(s * rpt, rpt)])
        plsc.subcore_barrier()

        def body(w, carry):
            base = (wid * jnp.int32(w_per) + w) * jnp.int32(C)
            pltpu.sync_copy(dstp.at[pl.ds(base, C)], dst_v)
            pltpu.sync_copy(ones_v, acc.at[dst_v], add=True)
            return carry

        lax.fori_loop(jnp.int32(0), jnp.int32(w_per), body, jnp.int32(0))
        plsc.subcore_barrier()
        pltpu.sync_copy(acc.at[pl.ds(s * rpt, rpt)], stg_v)
        pltpu.sync_copy(
            stg_v, out.at[pl.ds(c * jnp.int32(n_out_pad) + s * rpt, rpt)])

    return k


def _iota16():
    return lax.iota(I32, 16)


@functools.cache
def _map_edges(n_pad, k_pad, e_pad, kk, tbl):
    w_map = k_pad // (NTILE * C)
    w_per = e_pad // (NW * C)
    mesh = plsc.VectorSubcoreMesh(core_axis_name="c", subcore_axis_name="s")
    rpt = n_pad // NTILE
    ktrash = kk * kk

    @functools.partial(
        pl.kernel,
        name="map_edges",
        out_type=(jax.ShapeDtypeStruct((e_pad,), I32),
                  jax.ShapeDtypeStruct((e_pad,), I32),
                  jax.ShapeDtypeStruct((e_pad,), I32),
                  jax.ShapeDtypeStruct((tbl,), I32)),
        mesh=mesh,
        scratch_types=[
            pltpu.VMEM((C,), I32),
            pltpu.VMEM((C,), I32),
            pltpu.VMEM((C,), I32),
            pltpu.VMEM((C,), I32),
            pltpu.VMEM((C,), I32),
            pltpu.VMEM((C,), I32),
            pltpu.VMEM((rpt,), I32),
            pltpu.VMEM_SHARED((n_pad,), I32),
        ],
    )
    def k(idxp, srcp, dstp, s1o, d1o, keyo, tblo,
          a_v, b_v, s_v, d_v, key_v, eid_v, stg_v, newid):
        c = lax.axis_index("c")
        s = lax.axis_index("s")
        wid = c * NTILE + s

        def ifill(i, carry):
            stg_v[pl.ds(i * 16, 16)] = jnp.full((16,), -1, I32)
            return carry
        lax.fori_loop(jnp.int32(0), jnp.int32(rpt // 16), ifill, jnp.int32(0))
        pltpu.sync_copy(stg_v, newid.at[pl.ds(s * rpt, rpt)])
        plsc.subcore_barrier()

        def mbody(w, carry):
            base = (s * jnp.int32(w_map) + w) * jnp.int32(C)
            pltpu.sync_copy(idxp.at[pl.ds(base, C)], a_v)
            for j in range(C // 16):
                eid_v[pl.ds(j * 16, 16)] = base + jnp.int32(j * 16) + _iota16()
            pltpu.sync_copy(eid_v, newid.at[a_v])
            return carry
        lax.fori_loop(jnp.int32(0), jnp.int32(w_map), mbody, jnp.int32(0))
        plsc.subcore_barrier()

        def body(w, carry):
            base = (wid * jnp.int32(w_per) + w) * jnp.int32(C)
            pltpu.sync_copy(srcp.at[pl.ds(base, C)], a_v)
            pltpu.sync_copy(dstp.at[pl.ds(base, C)], b_v)
            pltpu.sync_copy(newid.at[a_v], s_v)
            pltpu.sync_copy(newid.at[b_v], d_v)
            for j in range(C // 16):
                sl = pl.ds(j * 16, 16)
                s16 = s_v[sl]
                d16 = d_v[sl]
                eid16 = base + jnp.int32(j * 16) + _iota16()
                m16 = (s16 >= 0) & (d16 >= 0)
                key16 = jnp.where(m16, s16 * jnp.int32(kk) + d16,
                                  jnp.int32(ktrash) + (eid16 & 2047))
                key_v[sl] = key16
                eid_v[sl] = eid16
            pltpu.sync_copy(s_v, s1o.at[pl.ds(base, C)])
            pltpu.sync_copy(d_v, d1o.at[pl.ds(base, C)])
            pltpu.sync_copy(key_v, keyo.at[pl.ds(base, C)])
            pltpu.sync_copy(eid_v, tblo.at[key_v])
            return carry
        lax.fori_loop(jnp.int32(0), jnp.int32(w_per), body, jnp.int32(0))

    return k


@functools.cache
def _finish_edges(pad_k, e_pad, kk):
    w_per = e_pad // (NW * C)
    mesh = plsc.VectorSubcoreMesh(core_axis_name="c", subcore_axis_name="s")
    rpt = pad_k // NTILE

    @functools.partial(
        pl.kernel,
        name="finish_edges",
        out_type=(jax.ShapeDtypeStruct((e_pad,), I32),
                  jax.ShapeDtypeStruct((e_pad,), I32),
                  jax.ShapeDtypeStruct((2 * pad_k,), F32)),
        mesh=mesh,
        scratch_types=[
            pltpu.VMEM((C,), I32),
            pltpu.VMEM((C,), I32),
            pltpu.VMEM((C,), I32),
            pltpu.VMEM((C,), I32),
            pltpu.VMEM((C,), I32),
            pltpu.VMEM((C,), I32),
            pltpu.VMEM((C,), F32),
            pltpu.VMEM((rpt,), F32),
            pltpu.VMEM_SHARED((pad_k,), F32),
            pltpu.SemaphoreType.DMA,
        ],
    )
    def k(s1, d1, key, tbl, srcfo, dstfo, dego,
          s_v, d_v, key_v, t_v, sf_v, df_v, ones_v, stg_v, acc, sem):
        c = lax.axis_index("c")
        s = lax.axis_index("s")
        wid = c * NTILE + s
        for i in range(C // 16):
            ones_v[pl.ds(i * 16, 16)] = jnp.full((16,), 1.0, F32)

        def zfill(i, carry):
            stg_v[pl.ds(i * 16, 16)] = jnp.zeros((16,), F32)
            return carry
        lax.fori_loop(jnp.int32(0), jnp.int32(rpt // 16), zfill, jnp.int32(0))
        pltpu.sync_copy(stg_v, acc.at[pl.ds(s * rpt, rpt)])
        plsc.subcore_barrier()

        def body(w, carry):
            base = (wid * jnp.int32(w_per) + w) * jnp.int32(C)
            pltpu.sync_copy(s1.at[pl.ds(base, C)], s_v)
            pltpu.sync_copy(d1.at[pl.ds(base, C)], d_v)
            pltpu.sync_copy(key.at[pl.ds(base, C)], key_v)
            pltpu.async_copy(tbl.at[key_v], t_v, sem).wait()
            for j in range(C // 16):
                sl = pl.ds(j * 16, 16)
                s16 = s_v[sl]
                d16 = d_v[sl]
                eid16 = base + jnp.int32(j * 16) + _iota16()
                ok = (s16 >= 0) & (d16 >= 0) & (t_v[sl] == eid16)
                sf_v[sl] = jnp.where(ok, s16, eid16 & 4095)
                df_v[sl] = jnp.where(ok, d16,
                                     jnp.int32(kk) + (eid16 & 63))
            pltpu.sync_copy(sf_v, srcfo.at[pl.ds(base, C)])
            pltpu.sync_copy(df_v, dstfo.at[pl.ds(base, C)])
            pltpu.sync_copy(ones_v, acc.at[df_v], add=True)
            return carry
        lax.fori_loop(jnp.int32(0), jnp.int32(w_per), body, jnp.int32(0))
        plsc.subcore_barrier()
        pltpu.sync_copy(acc.at[pl.ds(s * rpt, rpt)], stg_v)
        pltpu.sync_copy(
            stg_v, dego.at[pl.ds(c * jnp.int32(pad_k) + s * rpt, rpt)])

    return k


def _pad_edges(src, dst, n_in, n_out):
    e = src.shape[0]
    e_pad = _round_up(e, NW * C)
    pad = e_pad - e
    i = jnp.arange(pad, dtype=I32)
    src_p = jnp.concatenate([src, i % jnp.int32(n_in)])
    dst_p = jnp.concatenate([dst, jnp.int32(n_out) + (i % 64)])
    return src_p, dst_p, e_pad


def _sinus_row(t, dim):
    half = dim // 2
    cst = math.log(10000.0) / (half - 1)
    freqs = jnp.exp(jnp.arange(half, dtype=F32) * (-cst))
    e = t[0].astype(F32) * freqs
    return jnp.concatenate([jnp.sin(e), jnp.cos(e)])


def kernel(noised_data, t, edge_index, W_in, b_in, W0, b0, W1, b1, W2, b2,
           p_w, p_b, gamma, beta, W_fc, b_fc):
    n = noised_data.shape[1]
    kk = n // 2
    d = W0.shape[1]
    ei = edge_index.astype(I32)
    src0, dst0 = ei[0], ei[1]
    e = src0.shape[0]

    pad_n = _round_up(n + 64, 1024)
    pad_k = _round_up(kk + 64, 1024)

    src0p, dst0p, e_pad = _pad_edges(src0, dst0, n, n)

    dp = _deg_pass(pad_n, e_pad)(dst0p)
    deg0 = dp[:n] + dp[pad_n:pad_n + n] + 1.0
    r0 = lax.rsqrt(deg0)

    x = noised_data[0] @ W_in + b_in
    temb = _sinus_row(t, d)
    const0 = temb @ W0[W_in.shape[1]:]
    H0 = x @ W0[: W_in.shape[1]] + const0
    Hs0 = H0 * r0[:, None]

    S = _edge_pass(pad_n, e_pad, d)(Hs0, src0p, dst0p)
    h0 = jax.nn.relu(r0[:, None] * (S[0, :n] + S[1, :n] + Hs0) + b0)

    y = (h0 @ p_w)[:, 0] + p_b[0]
    topv, idx = lax.top_k(y, kk)
    idx32 = idx.astype(I32)
    yh = jax.nn.sigmoid(topv)
    x1 = h0[idx32] * yh[:, None]

    k_pad = _round_up(kk, NTILE * C)
    idxp = jnp.concatenate([
        idx32,
        jnp.int32(n + 64) + (jnp.arange(k_pad - kk, dtype=I32) % 64)])
    tbl = kk * kk + 2048
    s1a, d1a, keya, tbla = _map_edges(pad_n, k_pad, e_pad, kk, tbl)(
        idxp, src0p, dst0p)
    s1p, d1p, dego1 = _finish_edges(pad_k, e_pad, kk)(s1a, d1a, keya, tbla)
    deg1 = dego1[:kk] + dego1[pad_k:pad_k + kk] + 1.0
    r1 = lax.rsqrt(deg1)
    H1 = x1 @ W1
    Hs1 = H1 * r1[:, None]
    S1 = _edge_pass(pad_k, e_pad, d)(Hs1, s1p, d1p)
    h1 = jax.nn.relu(r1[:, None] * (S1[0, :kk] + S1[1, :kk] + Hs1) + b1)

    A = h0 @ W2
    B = h1 @ W2
    usrc, udst, ue_pad = _pad_edges(jnp.arange(kk, dtype=I32), idx32, kk, n)
    SU = _edge_pass(pad_n, ue_pad, d)(B, usrc, udst)
    U2 = SU[0, :n] + SU[1, :n] + A
    Hs2 = U2 * r0[:, None]

    S2 = _edge_pass(pad_n, e_pad, d)(Hs2, src0p, dst0p)
    h2 = jax.nn.relu(r0[:, None] * (S2[0, :n] + S2[1, :n] + Hs2) + b2)

    mu = jnp.mean(h2, axis=-1, keepdims=True)
    var = jnp.mean((h2 - mu) ** 2, axis=-1, keepdims=True)
    h2 = (h2 - mu) / jnp.sqrt(var + 1e-5) * gamma + beta
    return (h2 @ W_fc + b_fc)[None, ...]


# speedup vs baseline: 15.7256x; 2.9104x over previous
import functools
import math

import jax
import jax.numpy as jnp
from jax import lax
from jax.experimental import pallas as pl
from jax.experimental.pallas import tpu as pltpu
from jax.experimental.pallas import tpu_sc as plsc

F32 = jnp.float32
I32 = jnp.int32
NW = 32
NTILE = 16
C = 128


def _round_up(x, m):
    return (x + m - 1) // m * m


CO = 64


@functools.cache
def _edge_pass(n_out_pad, e_pad, d):
    w_per = e_pad // (NW * C)
    mesh = plsc.VectorSubcoreMesh(core_axis_name="c", subcore_axis_name="s")
    rpt = n_out_pad // NTILE
    assert rpt % CO == 0

    @functools.partial(
        pl.kernel,
        name="edge_pass",
        out_type=jax.ShapeDtypeStruct((2, n_out_pad, d), F32),
        mesh=mesh,
        scratch_types=[
            pltpu.VMEM((C,), I32),
            pltpu.VMEM((C,), I32),
            pltpu.VMEM((C, d), F32),
            pltpu.VMEM((CO, d), F32),
            pltpu.VMEM_SHARED((n_out_pad, d), F32),
            pltpu.SemaphoreType.DMA,
        ],
    )
    def k(hs, srcp, dstp, out, src_v, dst_v, rows_v, stg_v, acc, sem):
        c = lax.axis_index("c")
        s = lax.axis_index("s")
        wid = c * NTILE + s

        def zfill(i, carry):
            stg_v[i // jnp.int32(d // 16),
                  pl.ds((i % jnp.int32(d // 16)) * 16, 16)] = (
                      jnp.zeros((16,), F32))
            return carry
        lax.fori_loop(jnp.int32(0), jnp.int32(CO * d // 16), zfill,
                      jnp.int32(0))

        def zcp(i, carry):
            pltpu.sync_copy(stg_v, acc.at[pl.ds(s * rpt + i * jnp.int32(CO),
                                                CO)])
            return carry
        lax.fori_loop(jnp.int32(0), jnp.int32(rpt // CO), zcp, jnp.int32(0))
        plsc.subcore_barrier()

        def body(w, carry):
            base = (wid * jnp.int32(w_per) + w) * jnp.int32(C)
            pltpu.sync_copy(srcp.at[pl.ds(base, C)], src_v)
            pltpu.sync_copy(dstp.at[pl.ds(base, C)], dst_v)
            pltpu.async_copy(hs.at[src_v], rows_v, sem).wait()
            pltpu.sync_copy(rows_v, acc.at[dst_v], add=True)
            return carry

        lax.fori_loop(jnp.int32(0), jnp.int32(w_per), body, jnp.int32(0))
        plsc.subcore_barrier()

        def ocp(i, carry):
            off = s * rpt + i * jnp.int32(CO)
            pltpu.sync_copy(acc.at[pl.ds(off, CO)], stg_v)
            pltpu.sync_copy(stg_v, out.at[c, pl.ds(off, CO)])
            return carry
        lax.fori_loop(jnp.int32(0), jnp.int32(rpt // CO), ocp, jnp.int32(0))

    return k


@functools.cache
def _deg_pass(n_out_pad, e_pad):
    w_per = e_pad // (NW * C)
    mesh = plsc.VectorSubcoreMesh(core_axis_name="c", subcore_axis_name="s")
    rpt = n_out_pad // NTILE

    @functools.partial(
        pl.kernel,
        name="deg_pass",
        out_type=jax.ShapeDtypeStruct((2 * n_out_pad,), F32),
        mesh=mesh,
        scratch_types=[
            pltpu.VMEM((C,), I32),
            pltpu.VMEM((C,), F32),
            pltpu.VMEM((rpt,), F32),
            pltpu.VMEM_SHARED((n_out_pad,), F32),
        ],
    )
    def k(dstp, out, dst_v, ones_v, stg_v, acc):
        c = lax.axis_index("c")
        s = lax.axis_index("s")
        wid = c * NTILE + s
        for i in range(C // 16):
            ones_v[pl.ds(i * 16, 16)] = jnp.full((16,), 1.0, F32)

        def zfill(i, carry):
            stg_v[pl.ds(i * 16, 16)] = jnp.zeros((16,), F32)
            return carry
        lax.fori_loop(jnp.int32(0), jnp.int32(rpt // 16), zfill, jnp.int32(0))
        pltpu.sync_copy(stg_v, acc.at[pl.ds(s * rpt, rpt)])
        plsc.subcore_barrier()

        def body(w, carry):
            base = (wid * jnp.int32(w_per) + w) * jnp.int32(C)
            pltpu.sync_copy(dstp.at[pl.ds(base, C)], dst_v)
            pltpu.sync_copy(ones_v, acc.at[dst_v], add=True)
            return carry

        lax.fori_loop(jnp.int32(0), jnp.int32(w_per), body, jnp.int32(0))
        plsc.subcore_barrier()
        pltpu.sync_copy(acc.at[pl.ds(s * rpt, rpt)], stg_v)
        pltpu.sync_copy(
            stg_v, out.at[pl.ds(c * jnp.int32(n_out_pad) + s * rpt, rpt)])

    return k


def _iota16():
    return lax.iota(I32, 16)


@functools.cache
def _map_edges(n_pad, k_pad, e_pad, kk, tbl):
    w_map = k_pad // (NTILE * C)
    w_per = e_pad // (NW * C)
    mesh = plsc.VectorSubcoreMesh(core_axis_name="c", subcore_axis_name="s")
    rpt = n_pad // NTILE
    ktrash = kk * kk

    @functools.partial(
        pl.kernel,
        name="map_edges",
        out_type=(jax.ShapeDtypeStruct((e_pad,), I32),
                  jax.ShapeDtypeStruct((e_pad,), I32),
                  jax.ShapeDtypeStruct((e_pad,), I32),
                  jax.ShapeDtypeStruct((tbl,), I32)),
        mesh=mesh,
        scratch_types=[
            pltpu.VMEM((C,), I32),
            pltpu.VMEM((C,), I32),
            pltpu.VMEM((C,), I32),
            pltpu.VMEM((C,), I32),
            pltpu.VMEM((C,), I32),
            pltpu.VMEM((C,), I32),
            pltpu.VMEM((rpt,), I32),
            pltpu.VMEM_SHARED((n_pad,), I32),
        ],
    )
    def k(idxp, srcp, dstp, s1o, d1o, keyo, tblo,
          a_v, b_v, s_v, d_v, key_v, eid_v, stg_v, newid):
        c = lax.axis_index("c")
        s = lax.axis_index("s")
        wid = c * NTILE + s

        def ifill(i, carry):
            stg_v[pl.ds(i * 16, 16)] = jnp.full((16,), -1, I32)
            return carry
        lax.fori_loop(jnp.int32(0), jnp.int32(rpt // 16), ifill, jnp.int32(0))
        pltpu.sync_copy(stg_v, newid.at[pl.ds(s * rpt, rpt)])
        plsc.subcore_barrier()

        def mbody(w, carry):
            base = (s * jnp.int32(w_map) + w) * jnp.int32(C)
            pltpu.sync_copy(idxp.at[pl.ds(base, C)], a_v)
            for j in range(C // 16):
                eid_v[pl.ds(j * 16, 16)] = base + jnp.int32(j * 16) + _iota16()
            pltpu.sync_copy(eid_v, newid.at[a_v])
            return carry
        lax.fori_loop(jnp.int32(0), jnp.int32(w_map), mbody, jnp.int32(0))
        plsc.subcore_barrier()

        def body(w, carry):
            base = (wid * jnp.int32(w_per) + w) * jnp.int32(C)
            pltpu.sync_copy(srcp.at[pl.ds(base, C)], a_v)
            pltpu.sync_copy(dstp.at[pl.ds(base, C)], b_v)
            pltpu.sync_copy(newid.at[a_v], s_v)
            pltpu.sync_copy(newid.at[b_v], d_v)
            for j in range(C // 16):
                sl = pl.ds(j * 16, 16)
                s16 = s_v[sl]
                d16 = d_v[sl]
                eid16 = base + jnp.int32(j * 16) + _iota16()
                m16 = (s16 >= 0) & (d16 >= 0)
                key16 = jnp.where(m16, s16 * jnp.int32(kk) + d16,
                                  jnp.int32(ktrash) + eid16)
                key_v[sl] = key16
                eid_v[sl] = eid16
            pltpu.sync_copy(s_v, s1o.at[pl.ds(base, C)])
            pltpu.sync_copy(d_v, d1o.at[pl.ds(base, C)])
            pltpu.sync_copy(key_v, keyo.at[pl.ds(base, C)])
            pltpu.sync_copy(eid_v, tblo.at[key_v])
            return carry
        lax.fori_loop(jnp.int32(0), jnp.int32(w_per), body, jnp.int32(0))

    return k


@functools.cache
def _finish_edges(pad_k, e_pad, kk):
    w_per = e_pad // (NW * C)
    mesh = plsc.VectorSubcoreMesh(core_axis_name="c", subcore_axis_name="s")
    rpt = pad_k // NTILE

    @functools.partial(
        pl.kernel,
        name="finish_edges",
        out_type=(jax.ShapeDtypeStruct((e_pad,), I32),
                  jax.ShapeDtypeStruct((e_pad,), I32),
                  jax.ShapeDtypeStruct((2 * pad_k,), F32)),
        mesh=mesh,
        scratch_types=[
            pltpu.VMEM((C,), I32),
            pltpu.VMEM((C,), I32),
            pltpu.VMEM((C,), I32),
            pltpu.VMEM((C,), I32),
            pltpu.VMEM((C,), I32),
            pltpu.VMEM((C,), I32),
            pltpu.VMEM((C,), F32),
            pltpu.VMEM((rpt,), F32),
            pltpu.VMEM_SHARED((pad_k,), F32),
            pltpu.SemaphoreType.DMA,
        ],
    )
    def k(s1, d1, key, tbl, srcfo, dstfo, dego,
          s_v, d_v, key_v, t_v, sf_v, df_v, ones_v, stg_v, acc, sem):
        c = lax.axis_index("c")
        s = lax.axis_index("s")
        wid = c * NTILE + s
        for i in range(C // 16):
            ones_v[pl.ds(i * 16, 16)] = jnp.full((16,), 1.0, F32)

        def zfill(i, carry):
            stg_v[pl.ds(i * 16, 16)] = jnp.zeros((16,), F32)
            return carry
        lax.fori_loop(jnp.int32(0), jnp.int32(rpt // 16), zfill, jnp.int32(0))
        pltpu.sync_copy(stg_v, acc.at[pl.ds(s * rpt, rpt)])
        plsc.subcore_barrier()

        def body(w, carry):
            base = (wid * jnp.int32(w_per) + w) * jnp.int32(C)
            pltpu.sync_copy(s1.at[pl.ds(base, C)], s_v)
            pltpu.sync_copy(d1.at[pl.ds(base, C)], d_v)
            pltpu.sync_copy(key.at[pl.ds(base, C)], key_v)
            pltpu.async_copy(tbl.at[key_v], t_v, sem).wait()
            for j in range(C // 16):
                sl = pl.ds(j * 16, 16)
                s16 = s_v[sl]
                d16 = d_v[sl]
                eid16 = base + jnp.int32(j * 16) + _iota16()
                ok = (s16 >= 0) & (d16 >= 0) & (t_v[sl] == eid16)
                sf_v[sl] = jnp.where(ok, s16, eid16 & 4095)
                df_v[sl] = jnp.where(ok, d16,
                                     jnp.int32(kk) + (eid16 & 1023))
            pltpu.sync_copy(sf_v, srcfo.at[pl.ds(base, C)])
            pltpu.sync_copy(df_v, dstfo.at[pl.ds(base, C)])
            pltpu.sync_copy(ones_v, acc.at[df_v], add=True)
            return carry
        lax.fori_loop(jnp.int32(0), jnp.int32(w_per), body, jnp.int32(0))
        plsc.subcore_barrier()
        pltpu.sync_copy(acc.at[pl.ds(s * rpt, rpt)], stg_v)
        pltpu.sync_copy(
            stg_v, dego.at[pl.ds(c * jnp.int32(pad_k) + s * rpt, rpt)])

    return k


def _pad_edges(src, dst, n_in, n_out):
    e = src.shape[0]
    e_pad = _round_up(e, NW * C)
    pad = e_pad - e
    i = jnp.arange(pad, dtype=I32)
    src_p = jnp.concatenate([src, i % jnp.int32(n_in)])
    dst_p = jnp.concatenate([dst, jnp.int32(n_out) + (i % 64)])
    return src_p, dst_p, e_pad


def _sinus_row(t, dim):
    half = dim // 2
    cst = math.log(10000.0) / (half - 1)
    freqs = jnp.exp(jnp.arange(half, dtype=F32) * (-cst))
    e = t[0].astype(F32) * freqs
    return jnp.concatenate([jnp.sin(e), jnp.cos(e)])


def kernel(noised_data, t, edge_index, W_in, b_in, W0, b0, W1, b1, W2, b2,
           p_w, p_b, gamma, beta, W_fc, b_fc):
    n = noised_data.shape[1]
    kk = n // 2
    d = W0.shape[1]
    ei = edge_index.astype(I32)
    src0, dst0 = ei[0], ei[1]
    e = src0.shape[0]

    pad_n = _round_up(n + 64, 1024)
    pad_k = _round_up(kk + 64, 1024)

    src0p, dst0p, e_pad = _pad_edges(src0, dst0, n, n)

    dp = _deg_pass(pad_n, e_pad)(dst0p)
    deg0 = dp[:n] + dp[pad_n:pad_n + n] + 1.0
    r0 = lax.rsqrt(deg0)

    x = noised_data[0] @ W_in + b_in
    temb = _sinus_row(t, d)
    const0 = temb @ W0[W_in.shape[1]:]
    H0 = x @ W0[: W_in.shape[1]] + const0
    Hs0 = H0 * r0[:, None]

    S = _edge_pass(pad_n, e_pad, d)(Hs0, src0p, dst0p)
    h0 = jax.nn.relu(r0[:, None] * (S[0, :n] + S[1, :n] + Hs0) + b0)

    y = (h0 @ p_w)[:, 0] + p_b[0]
    topv, idx = lax.top_k(y, kk)
    idx32 = idx.astype(I32)
    yh = jax.nn.sigmoid(topv)
    x1 = h0[idx32] * yh[:, None]

    k_pad = _round_up(kk, NTILE * C)
    idxp = jnp.concatenate([
        idx32,
        jnp.int32(n + 64) + (jnp.arange(k_pad - kk, dtype=I32) % 64)])
    tbl = kk * kk + e_pad
    s1a, d1a, keya, tbla = _map_edges(pad_n, k_pad, e_pad, kk, tbl)(
        idxp, src0p, dst0p)
    s1p, d1p, dego1 = _finish_edges(pad_k, e_pad, kk)(s1a, d1a, keya, tbla)
    deg1 = dego1[:kk] + dego1[pad_k:pad_k + kk] + 1.0
    r1 = lax.rsqrt(deg1)
    H1 = x1 @ W1
    Hs1 = H1 * r1[:, None]
    S1 = _edge_pass(pad_k, e_pad, d)(Hs1, s1p, d1p)
    h1 = jax.nn.relu(r1[:, None] * (S1[0, :kk] + S1[1, :kk] + Hs1) + b1)

    A = h0 @ W2
    B = h1 @ W2
    usrc, udst, ue_pad = _pad_edges(jnp.arange(kk, dtype=I32), idx32, kk, n)
    SU = _edge_pass(pad_n, ue_pad, d)(B, usrc, udst)
    U2 = SU[0, :n] + SU[1, :n] + A
    Hs2 = U2 * r0[:, None]

    S2 = _edge_pass(pad_n, e_pad, d)(Hs2, src0p, dst0p)
    h2 = jax.nn.relu(r0[:, None] * (S2[0, :n] + S2[1, :n] + Hs2) + b2)

    mu = jnp.mean(h2, axis=-1, keepdims=True)
    var = jnp.mean((h2 - mu) ** 2, axis=-1, keepdims=True)
    h2 = (h2 - mu) / jnp.sqrt(var + 1e-5) * gamma + beta
    return (h2 @ W_fc + b_fc)[None, ...]
